# baseline (speedup 1.0000x reference)
import jax
import jax.numpy as jnp
from jax import lax
from jax.experimental import pallas as pl
from jax.experimental.pallas import tpu as pltpu
from jax.experimental.pallas import tpu_sc as plsc

_NAMES = ["f%d" % i for i in range(26)] + ["grp"]
_NF = 27
_NPLAIN = 26
_GRP_COLS = 4
_B = 4096
_EMB = 64
_NC = 2
_NS = 16
_NW = _NC * _NS
_BPW = _B // _NW
_NBUF = 4
_LAG = 2
_NQ = 2


def _enqueue_field_gather(tab, idx_row, dst, sems):
    def chunk(c, carry):
        v = idx_row[pl.ds(c * 16, 16)]
        for l in range(16):
            r = v[l]
            pltpu.async_copy(tab.at[pl.ds(r, 1)],
                             dst.at[pl.ds(c * 16 + l, 1)],
                             sems[l % _NQ])
        return carry

    lax.fori_loop(0, _BPW // 16, chunk, 0)


def _drain(tab, dst, sems):
    part = _BPW // _NQ
    for q in range(_NQ):
        pltpu.make_async_copy(tab.at[pl.ds(0, part)],
                              dst.at[pl.ds(q * part, part)], sems[q]).wait()


def _body(obs_hbm, *refs):
    tabs = refs[:_NF]
    outs = refs[_NF:2 * _NF]
    scratch = refs[2 * _NF:]
    idx_v = scratch[0]
    acc_v = scratch[1]
    rows = scratch[2:2 + _NBUF]
    gsems = [scratch[2 + _NBUF + b * _NQ:2 + _NBUF + (b + 1) * _NQ]
             for b in range(_NBUF)]
    wsems = scratch[2 + _NBUF + _NBUF * _NQ:]

    wid = lax.axis_index("s") * _NC + lax.axis_index("c")
    base = wid * _BPW

    pltpu.sync_copy(obs_hbm.at[:, pl.ds(base, _BPW)], idx_v)

    wcop = [None] * _NBUF
    nfields = _NPLAIN + _GRP_COLS
    for f in range(nfields + _LAG):
        if f < nfields:
            b = f % _NBUF
            if wcop[b] is not None:
                wcop[b].wait()
                wcop[b] = None
            tab = tabs[f] if f < _NPLAIN else tabs[_NF - 1]
            irow = f if f < _NPLAIN else f - _NPLAIN
            _enqueue_field_gather(tab, idx_v.at[irow], rows[b], gsems[b])
        d = f - _LAG
        if d >= 0:
            db = d % _NBUF
            dtab = tabs[d] if d < _NPLAIN else tabs[_NF - 1]
            _drain(dtab, rows[db], gsems[db])
            if d < _NPLAIN:
                wcop[db] = pltpu.async_copy(rows[db],
                                            outs[d].at[pl.ds(base, _BPW)],
                                            wsems[db])

    ga = rows[26 % _NBUF]
    gb = rows[27 % _NBUF]
    gc = rows[28 % _NBUF]
    gd = rows[29 % _NBUF]

    def _red(r, carry):
        for c in range(_EMB // 16):
            s0 = ga[r, pl.ds(c * 16, 16)]
            s1 = gb[r, pl.ds(c * 16, 16)]
            s2 = gc[r, pl.ds(c * 16, 16)]
            s3 = gd[r, pl.ds(c * 16, 16)]
            acc_v[r, pl.ds(c * 16, 16)] = (s0 + s1) + (s2 + s3)
        return carry

    lax.fori_loop(0, _BPW, _red, 0, unroll=4)

    pltpu.sync_copy(acc_v, outs[_NF - 1].at[pl.ds(base, _BPW)])
    for c in wcop:
        if c is not None:
            c.wait()


def kernel(observation, tables):
    obs_t = observation.T

    mesh = plsc.VectorSubcoreMesh(core_axis_name="c", subcore_axis_name="s")
    out_type = [jax.ShapeDtypeStruct((_B, _EMB), jnp.float32)] * _NF
    scratch = (
        [pltpu.VMEM((_NF, _BPW), jnp.int32),
         pltpu.VMEM((_BPW, _EMB), jnp.float32)]
        + [pltpu.VMEM((_BPW, _EMB), jnp.float32) for _ in range(_NBUF)]
        + [pltpu.SemaphoreType.DMA for _ in range(_NBUF * _NQ)]
        + [pltpu.SemaphoreType.DMA for _ in range(_NBUF)]
    )
    run = pl.kernel(_body, out_type=out_type, mesh=mesh,
                    scratch_types=scratch)
    outs = run(obs_t, *[tables[n] for n in _NAMES])
    return tuple(outs)
